# baseline (device time: 354348 ns/iter reference)
import jax
import jax.numpy as jnp
from jax import lax
from jax.experimental import pallas as pl
from jax.experimental.pallas import tpu as pltpu

N_DEV = 4
M_BLK = 2048
K_PER = 2048
K_GLB = 8192
N_OUT = 4096


def _a2a_body(x_ref, out_ref, local_sem, send_sems, recv_sems):
    my = lax.axis_index("i")

    bar = pltpu.get_barrier_semaphore()
    for off in (1, 2, 3):
        pl.semaphore_signal(
            bar, inc=1,
            device_id=((my + off) % N_DEV,),
            device_id_type=pl.DeviceIdType.MESH,
        )
    pl.semaphore_wait(bar, N_DEV - 1)

    local_copy = pltpu.make_async_copy(
        x_ref.at[pl.ds(my * M_BLK, M_BLK), :],
        out_ref.at[:, pl.ds(my * K_PER, K_PER)],
        local_sem,
    )
    local_copy.start()

    sends = []
    for off in (1, 2, 3):
        dst = (my + off) % N_DEV
        rdma = pltpu.make_async_remote_copy(
            src_ref=x_ref.at[pl.ds(dst * M_BLK, M_BLK), :],
            dst_ref=out_ref.at[:, pl.ds(my * K_PER, K_PER)],
            send_sem=send_sems.at[off - 1],
            recv_sem=recv_sems.at[off - 1],
            device_id=(dst,),
            device_id_type=pl.DeviceIdType.MESH,
        )
        rdma.start()
        sends.append(rdma)

    for off in (1, 2, 3):
        src = (my - off) % N_DEV
        recv = pltpu.make_async_remote_copy(
            src_ref=x_ref.at[pl.ds(0, M_BLK), :],
            dst_ref=out_ref.at[:, pl.ds(src * K_PER, K_PER)],
            send_sem=send_sems.at[off - 1],
            recv_sem=recv_sems.at[off - 1],
            device_id=(src,),
            device_id_type=pl.DeviceIdType.MESH,
        )
        recv.wait_recv()

    for rdma in sends:
        rdma.wait_send()
    local_copy.wait()


def _a2a(x_shard):
    return pl.pallas_call(
        _a2a_body,
        out_shape=jax.ShapeDtypeStruct((M_BLK, K_GLB), x_shard.dtype),
        in_specs=[pl.BlockSpec(memory_space=pl.ANY)],
        out_specs=pl.BlockSpec(memory_space=pl.ANY),
        scratch_shapes=[
            pltpu.SemaphoreType.DMA,
            pltpu.SemaphoreType.DMA((3,)),
            pltpu.SemaphoreType.DMA((3,)),
        ],
        compiler_params=pltpu.CompilerParams(collective_id=0),
    )(x_shard)


BM, BK, BN = 512, 2048, 1024
NM, NK, NN = M_BLK // BM, K_GLB // BK, N_OUT // BN


def _mm_body(xg_ref, w_ref, o_ref, acc_ref):
    k = pl.program_id(2)

    @pl.when(k == 0)
    def _():
        acc_ref[...] = jnp.zeros_like(acc_ref)

    acc_ref[...] += jnp.dot(
        xg_ref[...], w_ref[...], preferred_element_type=jnp.float32
    )

    @pl.when(k == NK - 1)
    def _():
        o_ref[...] = jnp.maximum(acc_ref[...], 0.0)


def _gemm_relu(xg, w_mat):
    return pl.pallas_call(
        _mm_body,
        grid=(NM, NN, NK),
        in_specs=[
            pl.BlockSpec((BM, BK), lambda i, j, k: (i, k)),
            pl.BlockSpec((BK, BN), lambda i, j, k: (k, j)),
        ],
        out_specs=pl.BlockSpec((BM, BN), lambda i, j, k: (i, j)),
        out_shape=jax.ShapeDtypeStruct((M_BLK, N_OUT), jnp.float32),
        scratch_shapes=[pltpu.VMEM((BM, BN), jnp.float32)],
        compiler_params=pltpu.CompilerParams(
            dimension_semantics=("parallel", "parallel", "arbitrary"),
        ),
    )(xg, w_mat)


BN2 = 256
NN2 = N_OUT // BN2
MH = M_BLK // 2
CT = 512
TPC = M_BLK // CT
_SLOT = {1: 1, 3: 2, 2: 3}
_CAST_ORDER = (1, 3, 2, 0)
NP4 = N_DEV + 1


def _fused_body(kb_ref, x_ref, w_ref, yin_ref, o_ref, x16_ref, xg_ref,
                stf_ref, stb_ref, in_sems, out_sems, send_sems, recv_sems):
    p = pl.program_id(0)
    j = pl.program_id(1)
    my = lax.axis_index("i")

    @pl.when((p == 0) & (j == 0))
    def _():
        bar = pltpu.get_barrier_semaphore()
        for off in (1, 2, 3):
            pl.semaphore_signal(
                bar, inc=1,
                device_id=((my + off) % N_DEV,),
                device_id_type=pl.DeviceIdType.MESH,
            )

        seq = [(off, t) for off in _CAST_ORDER for t in range(TPC)]
        in_dmas = []
        for i, (off, t) in enumerate(seq):
            chip = (my + off) % N_DEV
            in_dmas.append(pltpu.make_async_copy(
                x_ref.at[pl.ds(chip * M_BLK + t * CT, CT), :],
                stf_ref.at[i % 2],
                in_sems.at[i % 2],
            ))
        in_dmas[0].start()
        in_dmas[1].start()

        out_dmas = {}
        barrier_waited = False
        for i, (off, t) in enumerate(seq):
            s = i % 2
            in_dmas[i].wait()
            if off == 0:
                xg_ref[0, t * CT:(t + 1) * CT, :] = (
                    stf_ref[s].astype(jnp.bfloat16)
                )
            else:
                chip = (my + off) % N_DEV
                stb_ref[s, :, :] = stf_ref[s].astype(jnp.bfloat16)
                od = pltpu.make_async_copy(
                    stb_ref.at[s],
                    x16_ref.at[pl.ds(chip * M_BLK + t * CT, CT), :],
                    out_sems.at[s],
                )
                od.start()
                out_dmas[i] = od
            if i + 2 < len(seq):
                in_dmas[i + 2].start()
            if off != 0 and t % 2 == 1:
                out_dmas[i - 1].wait()
                out_dmas[i].wait()
                if not barrier_waited:
                    pl.semaphore_wait(bar, N_DEV - 1)
                    barrier_waited = True
                chip = (my + off) % N_DEV
                if off in (1, 3):
                    h = t // 2
                    slot = _SLOT[off]
                    rdma = pltpu.make_async_remote_copy(
                        src_ref=x16_ref.at[pl.ds(chip * M_BLK + h * MH, MH), :],
                        dst_ref=xg_ref.at[slot, pl.ds(h * MH, MH), :],
                        send_sem=send_sems.at[(off - 1) * 2 + h],
                        recv_sem=recv_sems.at[(slot - 1) * 2 + h],
                        device_id=(chip,),
                        device_id_type=pl.DeviceIdType.MESH,
                    )
                    rdma.start()
                elif t == TPC - 1:
                    for k in (0, 1):
                        rdma = pltpu.make_async_remote_copy(
                            src_ref=x16_ref.at[pl.ds(chip * M_BLK, M_BLK),
                                               pl.ds(k * MH, MH)],
                            dst_ref=xg_ref.at[3, :, pl.ds(k * MH, MH)],
                            send_sem=send_sems.at[2 + k],
                            recv_sem=recv_sems.at[4 + k],
                            device_id=(chip,),
                            device_id_type=pl.DeviceIdType.MESH,
                        )
                        rdma.start()

    for slot in (1, 2):
        @pl.when((p == slot) & (j == 0))
        def _(slot=slot):
            for h in (0, 1):
                recv = pltpu.make_async_remote_copy(
                    src_ref=x16_ref.at[pl.ds(0, MH), :],
                    dst_ref=xg_ref.at[slot, pl.ds(h * MH, MH), :],
                    send_sem=send_sems.at[0],
                    recv_sem=recv_sems.at[(slot - 1) * 2 + h],
                    device_id=(my,),
                    device_id_type=pl.DeviceIdType.MESH,
                )
                recv.wait_recv()

    for k in (0, 1):
        @pl.when((p == 3 + k) & (j == 0))
        def _(k=k):
            recv = pltpu.make_async_remote_copy(
                src_ref=x16_ref.at[pl.ds(0, M_BLK), pl.ds(0, MH)],
                dst_ref=xg_ref.at[3, :, pl.ds(k * MH, MH)],
                send_sem=send_sems.at[0],
                recv_sem=recv_sems.at[4 + k],
                device_id=(my,),
                device_id_type=pl.DeviceIdType.MESH,
            )
            recv.wait_recv()

    b = w_ref[...].astype(jnp.bfloat16)
    for s in range(NP4):
        @pl.when(p == s)
        def _(s=s):
            if s < 3:
                contrib = jnp.dot(
                    xg_ref[s], b, preferred_element_type=jnp.float32
                )
            else:
                k = s - 3
                contrib = jnp.dot(
                    xg_ref[3, :, k * MH:(k + 1) * MH],
                    b[k * MH:(k + 1) * MH, :],
                    preferred_element_type=jnp.float32,
                )
            if s == 0:
                o_ref[...] = contrib
            elif s < NP4 - 1:
                o_ref[...] = yin_ref[...] + contrib
            else:
                o_ref[...] = jnp.maximum(yin_ref[...] + contrib, 0.0)

    @pl.when((p == NP4 - 1) & (j == NN2 - 1))
    def _():
        for i in (0, 1, 4, 5):
            rdma = pltpu.make_async_remote_copy(
                src_ref=x16_ref.at[pl.ds(0, MH), :],
                dst_ref=xg_ref.at[1, pl.ds(0, MH), :],
                send_sem=send_sems.at[i],
                recv_sem=recv_sems.at[0],
                device_id=(my,),
                device_id_type=pl.DeviceIdType.MESH,
            )
            rdma.wait_send()
        for i in (2, 3):
            rdma = pltpu.make_async_remote_copy(
                src_ref=x16_ref.at[pl.ds(0, M_BLK), pl.ds(0, MH)],
                dst_ref=xg_ref.at[3, :, pl.ds(0, MH)],
                send_sem=send_sems.at[i],
                recv_sem=recv_sems.at[0],
                device_id=(my,),
                device_id_type=pl.DeviceIdType.MESH,
            )
            rdma.wait_send()


def _fused(x, w_mat, kb):
    grid_spec = pltpu.PrefetchScalarGridSpec(
        num_scalar_prefetch=1,
        grid=(NP4, NN2),
        in_specs=[
            pl.BlockSpec(memory_space=pl.ANY),
            pl.BlockSpec((K_PER, BN2), lambda p, j, kb_ref: (kb_ref[p], j)),
            pl.BlockSpec((M_BLK, BN2), lambda p, j, kb_ref: (0, j)),
        ],
        out_specs=[
            pl.BlockSpec((M_BLK, BN2), lambda p, j, kb_ref: (0, j)),
            pl.BlockSpec(memory_space=pl.ANY),
        ],
        scratch_shapes=[
            pltpu.VMEM((N_DEV, M_BLK, K_PER), jnp.bfloat16),
            pltpu.VMEM((2, CT, K_PER), jnp.float32),
            pltpu.VMEM((2, CT, K_PER), jnp.bfloat16),
            pltpu.SemaphoreType.DMA((2,)),
            pltpu.SemaphoreType.DMA((2,)),
            pltpu.SemaphoreType.DMA((6,)),
            pltpu.SemaphoreType.DMA((6,)),
        ],
    )
    y0 = jnp.zeros((M_BLK, N_OUT), jnp.float32)
    y, _ = pl.pallas_call(
        _fused_body,
        grid_spec=grid_spec,
        out_shape=[
            jax.ShapeDtypeStruct((M_BLK, N_OUT), jnp.float32),
            jax.ShapeDtypeStruct((K_GLB, K_PER), jnp.bfloat16),
        ],
        input_output_aliases={3: 0},
        compiler_params=pltpu.CompilerParams(
            dimension_semantics=("arbitrary", "arbitrary"),
            collective_id=0,
            vmem_limit_bytes=63 * 1024 * 1024,
        ),
    )(kb, x, w_mat, y0)
    return y


def kernel(x, w_mat):
    my = lax.axis_index("i")
    kb = (jnp.array([0, 3, 1, 2, 2], jnp.int32) + my) % N_DEV
    return _fused(x, w_mat, kb)


# device time: 325973 ns/iter; 1.0870x vs baseline; 1.0870x over previous
import jax
import jax.numpy as jnp
from jax import lax
from jax.experimental import pallas as pl
from jax.experimental.pallas import tpu as pltpu

N_DEV = 4
M_BLK = 2048
K_PER = 2048
K_GLB = 8192
N_OUT = 4096


def _a2a_body(x_ref, out_ref, local_sem, send_sems, recv_sems):
    my = lax.axis_index("i")

    bar = pltpu.get_barrier_semaphore()
    for off in (1, 2, 3):
        pl.semaphore_signal(
            bar, inc=1,
            device_id=((my + off) % N_DEV,),
            device_id_type=pl.DeviceIdType.MESH,
        )
    pl.semaphore_wait(bar, N_DEV - 1)

    local_copy = pltpu.make_async_copy(
        x_ref.at[pl.ds(my * M_BLK, M_BLK), :],
        out_ref.at[:, pl.ds(my * K_PER, K_PER)],
        local_sem,
    )
    local_copy.start()

    sends = []
    for off in (1, 2, 3):
        dst = (my + off) % N_DEV
        rdma = pltpu.make_async_remote_copy(
            src_ref=x_ref.at[pl.ds(dst * M_BLK, M_BLK), :],
            dst_ref=out_ref.at[:, pl.ds(my * K_PER, K_PER)],
            send_sem=send_sems.at[off - 1],
            recv_sem=recv_sems.at[off - 1],
            device_id=(dst,),
            device_id_type=pl.DeviceIdType.MESH,
        )
        rdma.start()
        sends.append(rdma)

    for off in (1, 2, 3):
        src = (my - off) % N_DEV
        recv = pltpu.make_async_remote_copy(
            src_ref=x_ref.at[pl.ds(0, M_BLK), :],
            dst_ref=out_ref.at[:, pl.ds(src * K_PER, K_PER)],
            send_sem=send_sems.at[off - 1],
            recv_sem=recv_sems.at[off - 1],
            device_id=(src,),
            device_id_type=pl.DeviceIdType.MESH,
        )
        recv.wait_recv()

    for rdma in sends:
        rdma.wait_send()
    local_copy.wait()


def _a2a(x_shard):
    return pl.pallas_call(
        _a2a_body,
        out_shape=jax.ShapeDtypeStruct((M_BLK, K_GLB), x_shard.dtype),
        in_specs=[pl.BlockSpec(memory_space=pl.ANY)],
        out_specs=pl.BlockSpec(memory_space=pl.ANY),
        scratch_shapes=[
            pltpu.SemaphoreType.DMA,
            pltpu.SemaphoreType.DMA((3,)),
            pltpu.SemaphoreType.DMA((3,)),
        ],
        compiler_params=pltpu.CompilerParams(collective_id=0),
    )(x_shard)


BM, BK, BN = 512, 2048, 1024
NM, NK, NN = M_BLK // BM, K_GLB // BK, N_OUT // BN


def _mm_body(xg_ref, w_ref, o_ref, acc_ref):
    k = pl.program_id(2)

    @pl.when(k == 0)
    def _():
        acc_ref[...] = jnp.zeros_like(acc_ref)

    acc_ref[...] += jnp.dot(
        xg_ref[...], w_ref[...], preferred_element_type=jnp.float32
    )

    @pl.when(k == NK - 1)
    def _():
        o_ref[...] = jnp.maximum(acc_ref[...], 0.0)


def _gemm_relu(xg, w_mat):
    return pl.pallas_call(
        _mm_body,
        grid=(NM, NN, NK),
        in_specs=[
            pl.BlockSpec((BM, BK), lambda i, j, k: (i, k)),
            pl.BlockSpec((BK, BN), lambda i, j, k: (k, j)),
        ],
        out_specs=pl.BlockSpec((BM, BN), lambda i, j, k: (i, j)),
        out_shape=jax.ShapeDtypeStruct((M_BLK, N_OUT), jnp.float32),
        scratch_shapes=[pltpu.VMEM((BM, BN), jnp.float32)],
        compiler_params=pltpu.CompilerParams(
            dimension_semantics=("parallel", "parallel", "arbitrary"),
        ),
    )(xg, w_mat)


BN2 = 256
NN2 = N_OUT // BN2
MH = M_BLK // 2
CT = 512
TPC = M_BLK // CT
_SLOT = {1: 1, 3: 2, 2: 3}
_CAST_ORDER = (1, 3, 2, 0)
NP4 = N_DEV + 1


def _fused_body(kb_ref, x_ref, w_ref, yin_ref, o_ref, x16_ref, xg_ref,
                stf_ref, stb_ref, in_sems, out_sems, send_sems, recv_sems):
    p = pl.program_id(0)
    j = pl.program_id(1)
    my = lax.axis_index("i")

    @pl.when((p == 0) & (j == 0))
    def _():
        bar = pltpu.get_barrier_semaphore()
        for off in (1, 2, 3):
            pl.semaphore_signal(
                bar, inc=1,
                device_id=((my + off) % N_DEV,),
                device_id_type=pl.DeviceIdType.MESH,
            )

        seq = [(off, t) for off in _CAST_ORDER for t in range(TPC)]
        in_dmas = []
        for i, (off, t) in enumerate(seq):
            chip = (my + off) % N_DEV
            in_dmas.append(pltpu.make_async_copy(
                x_ref.at[pl.ds(chip * M_BLK + t * CT, CT), :],
                stf_ref.at[i % 2],
                in_sems.at[i % 2],
            ))
        in_dmas[0].start()
        in_dmas[1].start()

        out_dmas = {}
        barrier_waited = False
        for i, (off, t) in enumerate(seq):
            s = i % 2
            in_dmas[i].wait()
            if off == 0:
                xg_ref[0, t * CT:(t + 1) * CT, :] = (
                    stf_ref[s].astype(jnp.bfloat16)
                )
            else:
                chip = (my + off) % N_DEV
                stb_ref[s, :, :] = stf_ref[s].astype(jnp.bfloat16)
                od = pltpu.make_async_copy(
                    stb_ref.at[s],
                    x16_ref.at[pl.ds(chip * M_BLK + t * CT, CT), :],
                    out_sems.at[s],
                )
                od.start()
                out_dmas[i] = od
            if i + 2 < len(seq):
                in_dmas[i + 2].start()
            if off != 0 and t % 2 == 1:
                out_dmas[i - 1].wait()
                out_dmas[i].wait()
                if not barrier_waited:
                    pl.semaphore_wait(bar, N_DEV - 1)
                    barrier_waited = True
                h = t // 2
                chip = (my + off) % N_DEV
                slot = _SLOT[off]
                rdma = pltpu.make_async_remote_copy(
                    src_ref=x16_ref.at[pl.ds(chip * M_BLK + h * MH, MH), :],
                    dst_ref=xg_ref.at[slot, pl.ds(h * MH, MH), :],
                    send_sem=send_sems.at[(off - 1) * 2 + h],
                    recv_sem=recv_sems.at[(slot - 1) * 2 + h],
                    device_id=(chip,),
                    device_id_type=pl.DeviceIdType.MESH,
                )
                rdma.start()

    for slot in (1, 2, 3):
        @pl.when((p == slot) & (j == 0))
        def _(slot=slot):
            for h in (0, 1):
                recv = pltpu.make_async_remote_copy(
                    src_ref=x16_ref.at[pl.ds(0, MH), :],
                    dst_ref=xg_ref.at[slot, pl.ds(h * MH, MH), :],
                    send_sem=send_sems.at[0],
                    recv_sem=recv_sems.at[(slot - 1) * 2 + h],
                    device_id=(my,),
                    device_id_type=pl.DeviceIdType.MESH,
                )
                recv.wait_recv()

    b = w_ref[...].astype(jnp.bfloat16)
    for s in range(N_DEV):
        @pl.when(p == s)
        def _(s=s):
            contrib = jnp.dot(
                xg_ref[s], b, preferred_element_type=jnp.float32
            )
            if s == 0:
                o_ref[...] = contrib
            elif s < N_DEV - 1:
                o_ref[...] = yin_ref[...] + contrib
            else:
                o_ref[...] = jnp.maximum(yin_ref[...] + contrib, 0.0)

    @pl.when((p == N_DEV - 1) & (j == NN2 - 1))
    def _():
        for i in range(6):
            rdma = pltpu.make_async_remote_copy(
                src_ref=x16_ref.at[pl.ds(0, MH), :],
                dst_ref=xg_ref.at[1, pl.ds(0, MH), :],
                send_sem=send_sems.at[i],
                recv_sem=recv_sems.at[0],
                device_id=(my,),
                device_id_type=pl.DeviceIdType.MESH,
            )
            rdma.wait_send()


def _fused(x, w_mat, kb):
    grid_spec = pltpu.PrefetchScalarGridSpec(
        num_scalar_prefetch=1,
        grid=(N_DEV, NN2),
        in_specs=[
            pl.BlockSpec(memory_space=pl.ANY),
            pl.BlockSpec((K_PER, BN2), lambda p, j, kb_ref: (kb_ref[p], j)),
            pl.BlockSpec((M_BLK, BN2), lambda p, j, kb_ref: (0, j)),
        ],
        out_specs=[
            pl.BlockSpec((M_BLK, BN2), lambda p, j, kb_ref: (0, j)),
            pl.BlockSpec(memory_space=pl.ANY),
        ],
        scratch_shapes=[
            pltpu.VMEM((N_DEV, M_BLK, K_PER), jnp.bfloat16),
            pltpu.VMEM((2, CT, K_PER), jnp.float32),
            pltpu.VMEM((2, CT, K_PER), jnp.bfloat16),
            pltpu.SemaphoreType.DMA((2,)),
            pltpu.SemaphoreType.DMA((2,)),
            pltpu.SemaphoreType.DMA((6,)),
            pltpu.SemaphoreType.DMA((6,)),
        ],
    )
    y0 = jnp.zeros((M_BLK, N_OUT), jnp.float32)
    y, _ = pl.pallas_call(
        _fused_body,
        grid_spec=grid_spec,
        out_shape=[
            jax.ShapeDtypeStruct((M_BLK, N_OUT), jnp.float32),
            jax.ShapeDtypeStruct((K_GLB, K_PER), jnp.bfloat16),
        ],
        input_output_aliases={3: 0},
        compiler_params=pltpu.CompilerParams(
            dimension_semantics=("arbitrary", "arbitrary"),
            collective_id=0,
            vmem_limit_bytes=63 * 1024 * 1024,
        ),
    )(kb, x, w_mat, y0)
    return y


def kernel(x, w_mat):
    my = lax.axis_index("i")
    kb = (jnp.array([0, 3, 1, 2], jnp.int32) + my) % N_DEV
    return _fused(x, w_mat, kb)
